# baseline (speedup 1.0000x reference)
import functools

import jax
import jax.numpy as jnp
from jax import lax
from jax.experimental import pallas as pl
from jax.experimental.pallas import tpu as pltpu
from jax.experimental.pallas import tpu_sc as plsc

BATCH = 16384
EMBED_DIM = 64
PAIRS = BATCH // 2
PAIR_DIM = 2 * EMBED_DIM

_info = plsc.get_sparse_core_info()
_NW = _info.num_cores * _info.num_subcores
_P_PER_W = PAIRS // _NW
_R_PER_W = BATCH // _NW
_IDX_CHUNK = 128
_LANES = 16


@functools.partial(
    pl.kernel,
    mesh=plsc.VectorSubcoreMesh(core_axis_name="c", subcore_axis_name="s"),
    out_type=jax.ShapeDtypeStruct((PAIRS, PAIR_DIM), jnp.float32),
    scratch_types=[
        pltpu.VMEM((_R_PER_W,), jnp.int32),
        pltpu.VMEM((_P_PER_W,), jnp.int32),
        pltpu.VMEM((_P_PER_W, PAIR_DIM), jnp.float32),
        pltpu.SemaphoreType.DMA,
    ],
    compiler_params=pltpu.CompilerParams(needs_layout_passes=False),
)
def _pair_lookup(table_hbm, role_hbm, out_hbm, role_v, pair_v, rows_v, sem):
    wid = lax.axis_index("s") * _info.num_cores + lax.axis_index("c")
    pltpu.sync_copy(role_hbm.at[pl.ds(wid * _R_PER_W, _R_PER_W)], role_v)

    lane = lax.iota(jnp.int32, _LANES)

    def pair_body(k, _):
        base = 2 * _LANES * k
        even = plsc.load_gather(role_v, [base + 2 * lane])
        odd = plsc.load_gather(role_v, [base + 2 * lane + 1])
        pair_v[pl.ds(_LANES * k, _LANES)] = 2 * even + odd
        return 0

    lax.fori_loop(0, _P_PER_W // _LANES, pair_body, 0)

    for j in range(_P_PER_W // _IDX_CHUNK):
        pltpu.async_copy(
            table_hbm.at[pair_v.at[pl.ds(j * _IDX_CHUNK, _IDX_CHUNK)]],
            rows_v.at[pl.ds(j * _IDX_CHUNK, _IDX_CHUNK)],
            sem,
        ).wait()

    pltpu.sync_copy(rows_v, out_hbm.at[pl.ds(wid * _P_PER_W, _P_PER_W)])


def kernel(obs, role, W_role):
    del obs
    table4 = jnp.concatenate(
        [jnp.repeat(W_role, 2, axis=0), jnp.tile(W_role, (2, 1))], axis=1
    )
    out_pairs = _pair_lookup(table4, role)
    return out_pairs.reshape(BATCH, EMBED_DIM)
